# baseline (speedup 1.0000x reference)
import dataclasses
import functools

import jax
import jax.numpy as jnp
from jax import lax
from jax.experimental import pallas as pl
from jax.experimental.pallas import tpu as pltpu
from jax.experimental.pallas import tpu_sc as plsc

N = 10000
E = 320000
H = 128
MAXZ = 100
NG = 64
NCONV = 3

NTILES = 32
CHUNK = 128
EPT = 10112
NCHUNK = EPT // CHUNK
EPAD = EPT * NTILES
SROWS = 10112
NOUT = 10112

_mesh = plsc.VectorSubcoreMesh(core_axis_name="c", subcore_axis_name="s")

_sc_params = pltpu.CompilerParams()
if "needs_layout_passes" in pltpu.CompilerParams.__dataclass_fields__:
    _sc_params = dataclasses.replace(_sc_params, needs_layout_passes=False)


def _matT(a, b):
    return lax.dot_general(a, b, (((1,), (1,)), ((), ())),
                           preferred_element_type=jnp.float32)


def _edge_body(a_hbm, b_hbm, rg_hbm, rs_hbm, cg_hbm, ea_hbm, w_hbm, out_hbm,
               rg_v, rs_v, cg_v, ea_v, a_buf, b_buf, w_v, zbuf, s_sh):
    ci = lax.axis_index("c")
    si = lax.axis_index("s")

    zeros16 = jnp.zeros((16,), jnp.float32)

    @pl.loop(0, 8)
    def _(r):
        for j in range(H // 16):
            zbuf[r, pl.ds(j * 16, 16)] = zeros16

    @pl.loop(0, SROWS // 16 // 8)
    def _(k):
        pltpu.sync_copy(zbuf, s_sh.at[pl.ds(si * (SROWS // 16) + k * 8, 8), :])

    pltpu.sync_copy(w_hbm, w_v)
    plsc.subcore_barrier()

    wblocks = [w_v[pl.ds(j * 16, 16)] for j in range(8)]
    wid = ci * 16 + si

    @pl.loop(0, NCHUNK)
    def _(k):
        j = wid * NCHUNK + k
        pltpu.sync_copy(rg_hbm.at[j], rg_v.at[0])
        pltpu.sync_copy(rs_hbm.at[j], rs_v.at[0])
        pltpu.sync_copy(cg_hbm.at[j], cg_v.at[0])
        pltpu.sync_copy(ea_hbm.at[j], ea_v)
        pltpu.sync_copy(a_hbm.at[rg_v.at[0]], a_buf)
        pltpu.sync_copy(b_hbm.at[cg_v.at[0]], b_buf)

        @pl.loop(0, CHUNK)
        def _(e):
            eidx = jnp.full((16,), e, jnp.int32)
            ea16 = plsc.load_gather(ea_v, [eidx])
            for jb in range(8):
                a = a_buf[e, pl.ds(jb * 16, 16)]
                b = b_buf[e, pl.ds(jb * 16, 16)]
                t = a + b + ea16 * wblocks[jb]
                a_buf[e, pl.ds(jb * 16, 16)] = t / (1.0 + jnp.exp(-t))

        pltpu.sync_copy(a_buf, s_sh.at[rs_v.at[0]], add=True)

    plsc.subcore_barrier()
    rows = NOUT // 16
    pltpu.sync_copy(s_sh.at[pl.ds(si * rows, rows), :],
                    out_hbm.at[ci, pl.ds(si * rows, rows), :])


_edge_kernel = functools.partial(
    pl.kernel,
    out_type=jax.ShapeDtypeStruct((2, NOUT, H), jnp.float32),
    mesh=_mesh,
    compiler_params=_sc_params,
    scratch_types=[
        pltpu.VMEM((1, CHUNK), jnp.int32),
        pltpu.VMEM((1, CHUNK), jnp.int32),
        pltpu.VMEM((1, CHUNK), jnp.int32),
        pltpu.VMEM((CHUNK,), jnp.float32),
        pltpu.VMEM((CHUNK, H), jnp.float32),
        pltpu.VMEM((CHUNK, H), jnp.float32),
        pltpu.VMEM((H,), jnp.float32),
        pltpu.VMEM((8, H), jnp.float32),
        pltpu.VMEM_SHARED((SROWS, H), jnp.float32),
    ],
)(_edge_body)


def _deg_body(rs_hbm, out_hbm, rs_v, ones_buf, zbuf, s_sh):
    ci = lax.axis_index("c")
    si = lax.axis_index("s")
    zeros16 = jnp.zeros((16,), jnp.float32)
    ones16 = jnp.ones((16,), jnp.float32)

    @pl.loop(0, CHUNK)
    def _(r):
        for j in range(H // 16):
            ones_buf[r, pl.ds(j * 16, 16)] = ones16

    @pl.loop(0, 8)
    def _(r):
        for j in range(H // 16):
            zbuf[r, pl.ds(j * 16, 16)] = zeros16

    @pl.loop(0, SROWS // 16 // 8)
    def _(k):
        pltpu.sync_copy(zbuf, s_sh.at[pl.ds(si * (SROWS // 16) + k * 8, 8), :])

    plsc.subcore_barrier()
    wid = ci * 16 + si

    @pl.loop(0, NCHUNK)
    def _(k):
        j = wid * NCHUNK + k
        pltpu.sync_copy(rs_hbm.at[j], rs_v.at[0])
        pltpu.sync_copy(ones_buf, s_sh.at[rs_v.at[0]], add=True)

    plsc.subcore_barrier()
    rows = NOUT // 16
    pltpu.sync_copy(s_sh.at[pl.ds(si * rows, rows), :],
                    out_hbm.at[ci, pl.ds(si * rows, rows), :])


_deg_kernel = functools.partial(
    pl.kernel,
    out_type=jax.ShapeDtypeStruct((2, NOUT, H), jnp.float32),
    mesh=_mesh,
    compiler_params=_sc_params,
    scratch_types=[
        pltpu.VMEM((1, CHUNK), jnp.int32),
        pltpu.VMEM((CHUNK, H), jnp.float32),
        pltpu.VMEM((8, H), jnp.float32),
        pltpu.VMEM_SHARED((SROWS, H), jnp.float32),
    ],
)(_deg_body)


def _tc0_body(x_ref, emb_ref, pw_ref, pb_ref, w1a_ref, w1b_ref, b1_ref,
              h_ref, a_ref, b_ref):
    oh = (x_ref[...] == lax.broadcasted_iota(jnp.int32, (N, MAXZ + 1), 1))
    oh = oh.astype(jnp.float32)
    he = jnp.dot(oh, emb_ref[...], preferred_element_type=jnp.float32)
    h = _matT(he, pw_ref[...]) + pb_ref[...]
    h_ref[...] = h
    a_ref[...] = _matT(h, w1a_ref[...]) + b1_ref[...]
    b_ref[...] = _matT(h, w1b_ref[...])


def _tc0(x, emb, pw, pb, w1a, w1b, b1):
    f32 = jnp.float32
    return pl.pallas_call(
        _tc0_body,
        out_shape=[jax.ShapeDtypeStruct((N, H), f32)] * 3,
    )(x, emb, pw, pb, w1a, w1b, b1)


def _gru_ln(s2, dg2, h, w2, b2, wih, whh, bih, bhh, lng, lnb):
    s = s2[0] + s2[1]
    deg = dg2[0] + dg2[1]
    agg = _matT(s, w2) + deg * b2
    gi = _matT(agg, wih) + bih
    gh = _matT(h, whh) + bhh
    r = jax.nn.sigmoid(gi[:, :H] + gh[:, :H])
    z = jax.nn.sigmoid(gi[:, H:2 * H] + gh[:, H:2 * H])
    n = jnp.tanh(gi[:, 2 * H:] + r * gh[:, 2 * H:])
    hn = h + (1.0 - z) * n + z * h
    mu = jnp.mean(hn, axis=1, keepdims=True)
    var = jnp.mean((hn - mu) ** 2, axis=1, keepdims=True)
    return (hn - mu) * lax.rsqrt(var + 1e-5) * lng + lnb


def _tc_mid_body(s2_ref, dg2_ref, h_ref, w2_ref, b2_ref, wih_ref, whh_ref,
                 bih_ref, bhh_ref, lng_ref, lnb_ref, w1a_ref, w1b_ref, b1_ref,
                 h_out, a_out, b_out):
    hl = _gru_ln(s2_ref[...], dg2_ref[...], h_ref[...], w2_ref[...],
                 b2_ref[...], wih_ref[...], whh_ref[...], bih_ref[...],
                 bhh_ref[...], lng_ref[...], lnb_ref[...])
    h_out[...] = hl
    a_out[...] = _matT(hl, w1a_ref[...]) + b1_ref[...]
    b_out[...] = _matT(hl, w1b_ref[...])


def _tc_last_body(s2_ref, dg2_ref, h_ref, w2_ref, b2_ref, wih_ref, whh_ref,
                  bih_ref, bhh_ref, lng_ref, lnb_ref, h_out):
    h_out[...] = _gru_ln(s2_ref[...], dg2_ref[...], h_ref[...], w2_ref[...],
                         b2_ref[...], wih_ref[...], whh_ref[...], bih_ref[...],
                         bhh_ref[...], lng_ref[...], lnb_ref[...])


_ROWB = 1000
_GRID = N // _ROWB


def _node_specs():
    full = lambda shape: pl.BlockSpec(shape, lambda i: tuple(0 for _ in shape))
    return [
        pl.BlockSpec((2, _ROWB, H), lambda i: (0, i, 0)),
        pl.BlockSpec((2, _ROWB, 1), lambda i: (0, i, 0)),
        pl.BlockSpec((_ROWB, H), lambda i: (i, 0)),
        full((H, H)), full((1, H)),
        full((3 * H, H)), full((3 * H, H)), full((1, 3 * H)), full((1, 3 * H)),
        full((1, H)), full((1, H)),
    ]


def _tc_mid(s2, dg2, h, w2, b2, wih, whh, bih, bhh, lng, lnb, w1a, w1b, b1):
    f32 = jnp.float32
    full = lambda shape: pl.BlockSpec(shape, lambda i: tuple(0 for _ in shape))
    row = pl.BlockSpec((_ROWB, H), lambda i: (i, 0))
    return pl.pallas_call(
        _tc_mid_body,
        grid=(_GRID,),
        in_specs=_node_specs() + [full((H, H)), full((H, H)), full((1, H))],
        out_specs=[row, row, row],
        out_shape=[jax.ShapeDtypeStruct((N, H), f32)] * 3,
    )(s2, dg2, h, w2, b2, wih, whh, bih, bhh, lng, lnb, w1a, w1b, b1)


def _tc_last(s2, dg2, h, w2, b2, wih, whh, bih, bhh, lng, lnb):
    f32 = jnp.float32
    row = pl.BlockSpec((_ROWB, H), lambda i: (i, 0))
    return pl.pallas_call(
        _tc_last_body,
        grid=(_GRID,),
        in_specs=_node_specs(),
        out_specs=row,
        out_shape=jax.ShapeDtypeStruct((N, H), f32),
    )(s2, dg2, h, w2, b2, wih, whh, bih, bhh, lng, lnb)


def _tc_fin(h, batch2, h1w, h1b, h2w, h2b):
    f32 = jnp.float32
    return pl.pallas_call(
        _fin_body2,
        out_shape=jax.ShapeDtypeStruct((NG, 1), f32),
    )(h, batch2, h1w, h1b, h2w, h2b)


def _fin_body2(h_ref, batch_ref, h1w_ref, h1b_ref, h2w_ref, h2b_ref, out_ref):
    oh = (batch_ref[...] == lax.broadcasted_iota(jnp.int32, (N, NG), 1))
    oh = oh.astype(jnp.float32)
    sums = lax.dot_general(oh, h_ref[...], (((0,), (0,)), ((), ())),
                           preferred_element_type=jnp.float32)
    ones = jnp.ones((N, 1), jnp.float32)
    cnts = lax.dot_general(oh, ones, (((0,), (0,)), ((), ())),
                           preferred_element_type=jnp.float32)
    g = sums / jnp.maximum(cnts, 1.0)
    t = _matT(g, h1w_ref[...]) + h1b_ref[...]
    g1 = t * jax.nn.sigmoid(t)
    out_ref[...] = (jnp.sum(g1 * h2w_ref[...], axis=1, keepdims=True)
                    + h2b_ref[...])


def kernel(x, edge_index, edge_attr, batch, emb, proj_W, proj_b, msg1_W,
           msg1_b, msg2_W, msg2_b, gru_Wih, gru_Whh, gru_bih, gru_bhh,
           ln_g, ln_b, head1_W, head1_b, head2_W, head2_b):
    f32 = jnp.float32
    row = edge_index[0].astype(jnp.int32)
    col = edge_index[1].astype(jnp.int32)
    pad = EPAD - E
    rg = jnp.concatenate([row, jnp.zeros((pad,), jnp.int32)]).reshape(-1, CHUNK)
    rs = jnp.concatenate([row, jnp.full((pad,), N, jnp.int32)]).reshape(-1, CHUNK)
    cg = jnp.concatenate([col, jnp.zeros((pad,), jnp.int32)]).reshape(-1, CHUNK)
    ea = jnp.concatenate([edge_attr[:, 0].astype(f32),
                          jnp.zeros((pad,), f32)]).reshape(-1, CHUNK)

    pb = proj_b.reshape(1, H)
    w1a = [msg1_W[l, :, :H] for l in range(NCONV)]
    w1b = [msg1_W[l, :, H:2 * H] for l in range(NCONV)]
    w1c = [msg1_W[l, :, 2 * H] for l in range(NCONV)]
    b1 = [msg1_b[l].reshape(1, H) for l in range(NCONV)]
    b2 = [msg2_b[l].reshape(1, H) for l in range(NCONV)]

    h, a, b = _tc0(x.astype(jnp.int32), emb, proj_W, pb, w1a[0], w1b[0], b1[0])
    dg2 = _deg_kernel(rs)[:, :N, 0:1]

    for l in range(NCONV):
        s2 = _edge_kernel(a, b, rg, rs, cg, ea, w1c[l])[:, :N]
        args = (s2, dg2, h, msg2_W[l], b2[l], gru_Wih[l], gru_Whh[l],
                gru_bih[l].reshape(1, 3 * H), gru_bhh[l].reshape(1, 3 * H),
                ln_g[l].reshape(1, H), ln_b[l].reshape(1, H))
        if l < NCONV - 1:
            h, a, b = _tc_mid(*args, w1a[l + 1], w1b[l + 1], b1[l + 1])
        else:
            h = _tc_last(*args)

    out2 = _tc_fin(h, batch.astype(jnp.int32).reshape(N, 1), head1_W,
                   head1_b.reshape(1, H), head2_W, head2_b.reshape(1, 1))
    return out2[:, 0]


# speedup vs baseline: 2.7340x; 2.7340x over previous
import dataclasses
import functools

import jax
import jax.numpy as jnp
from jax import lax
from jax.experimental import pallas as pl
from jax.experimental.pallas import tpu as pltpu
from jax.experimental.pallas import tpu_sc as plsc

N = 10000
E = 320000
H = 128
MAXZ = 100
NG = 64
NCONV = 3

NTILES = 32
CHUNK = 128
EPT = 10112
NCHUNK = EPT // CHUNK
EPAD = EPT * NTILES
SROWS = 10112
NOUT = 10112

_mesh = plsc.VectorSubcoreMesh(core_axis_name="c", subcore_axis_name="s")

_sc_params = pltpu.CompilerParams()
if "needs_layout_passes" in pltpu.CompilerParams.__dataclass_fields__:
    _sc_params = dataclasses.replace(_sc_params, needs_layout_passes=False)


def _matT(a, b):
    return lax.dot_general(a, b, (((1,), (1,)), ((), ())),
                           preferred_element_type=jnp.float32)


def _edge_body(a_hbm, b_hbm, idx_hbm, w_hbm, out_hbm,
               idx_v, a_buf, b_buf, w_v, zbuf, s_sh):
    ci = lax.axis_index("c")
    si = lax.axis_index("s")

    zeros16 = jnp.zeros((16,), jnp.float32)

    @pl.loop(0, 8)
    def _(r):
        for j in range(H // 16):
            zbuf[r, pl.ds(j * 16, 16)] = zeros16

    @pl.loop(0, SROWS // 16 // 8)
    def _(k):
        pltpu.sync_copy(zbuf, s_sh.at[pl.ds(si * (SROWS // 16) + k * 8, 8), :])

    pltpu.sync_copy(w_hbm, w_v)
    plsc.subcore_barrier()

    wblocks = [w_v[pl.ds(j * 16, 16)] for j in range(8)]
    wid = ci * 16 + si

    three16 = jnp.full((16,), 3, jnp.int32)

    @pl.loop(0, NCHUNK)
    def _(k):
        j = wid * NCHUNK + k
        pltpu.sync_copy(idx_hbm.at[j], idx_v)
        pltpu.sync_copy(a_hbm.at[idx_v.at[0]], a_buf)
        pltpu.sync_copy(b_hbm.at[idx_v.at[2]], b_buf)

        @plsc.parallel_loop(0, CHUNK, unroll=4)
        def _(e):
            eidx = jnp.full((16,), e, jnp.int32)
            ea16 = plsc.bitcast(plsc.load_gather(idx_v, [three16, eidx]),
                                jnp.float32)
            for jb in range(8):
                a = a_buf[e, pl.ds(jb * 16, 16)]
                b = b_buf[e, pl.ds(jb * 16, 16)]
                t = a + b + ea16 * wblocks[jb]
                a_buf[e, pl.ds(jb * 16, 16)] = t / (1.0 + jnp.exp(-t))

        pltpu.sync_copy(a_buf, s_sh.at[idx_v.at[1]], add=True)

    plsc.subcore_barrier()
    rows = NOUT // 16
    pltpu.sync_copy(s_sh.at[pl.ds(si * rows, rows), :],
                    out_hbm.at[ci, pl.ds(si * rows, rows), :])


_edge_kernel = functools.partial(
    pl.kernel,
    out_type=jax.ShapeDtypeStruct((2, NOUT, H), jnp.float32),
    mesh=_mesh,
    compiler_params=_sc_params,
    scratch_types=[
        pltpu.VMEM((4, CHUNK), jnp.int32),
        pltpu.VMEM((CHUNK, H), jnp.float32),
        pltpu.VMEM((CHUNK, H), jnp.float32),
        pltpu.VMEM((H,), jnp.float32),
        pltpu.VMEM((8, H), jnp.float32),
        pltpu.VMEM_SHARED((SROWS, H), jnp.float32),
    ],
)(_edge_body)


def _deg_body(idx_hbm, out_hbm, idx_v, ones_buf, zbuf, s_sh):
    ci = lax.axis_index("c")
    si = lax.axis_index("s")
    zeros16 = jnp.zeros((16,), jnp.float32)
    ones16 = jnp.ones((16,), jnp.float32)

    @pl.loop(0, CHUNK)
    def _(r):
        for j in range(H // 16):
            ones_buf[r, pl.ds(j * 16, 16)] = ones16

    @pl.loop(0, 8)
    def _(r):
        for j in range(H // 16):
            zbuf[r, pl.ds(j * 16, 16)] = zeros16

    @pl.loop(0, SROWS // 16 // 8)
    def _(k):
        pltpu.sync_copy(zbuf, s_sh.at[pl.ds(si * (SROWS // 16) + k * 8, 8), :])

    plsc.subcore_barrier()
    wid = ci * 16 + si

    @pl.loop(0, NCHUNK)
    def _(k):
        j = wid * NCHUNK + k
        pltpu.sync_copy(idx_hbm.at[j], idx_v)
        pltpu.sync_copy(ones_buf, s_sh.at[idx_v.at[1]], add=True)

    plsc.subcore_barrier()
    rows = NOUT // 16
    pltpu.sync_copy(s_sh.at[pl.ds(si * rows, rows), :],
                    out_hbm.at[ci, pl.ds(si * rows, rows), :])


_deg_kernel = functools.partial(
    pl.kernel,
    out_type=jax.ShapeDtypeStruct((2, NOUT, H), jnp.float32),
    mesh=_mesh,
    compiler_params=_sc_params,
    scratch_types=[
        pltpu.VMEM((4, CHUNK), jnp.int32),
        pltpu.VMEM((CHUNK, H), jnp.float32),
        pltpu.VMEM((8, H), jnp.float32),
        pltpu.VMEM_SHARED((SROWS, H), jnp.float32),
    ],
)(_deg_body)


def _tc0_body(x_ref, emb_ref, pw_ref, pb_ref, w1a_ref, w1b_ref, b1_ref,
              h_ref, a_ref, b_ref):
    oh = (x_ref[...] == lax.broadcasted_iota(jnp.int32, (N, MAXZ + 1), 1))
    oh = oh.astype(jnp.float32)
    he = jnp.dot(oh, emb_ref[...], preferred_element_type=jnp.float32)
    h = _matT(he, pw_ref[...]) + pb_ref[...]
    h_ref[...] = h
    a_ref[...] = _matT(h, w1a_ref[...]) + b1_ref[...]
    b_ref[...] = _matT(h, w1b_ref[...])


def _tc0(x, emb, pw, pb, w1a, w1b, b1):
    f32 = jnp.float32
    return pl.pallas_call(
        _tc0_body,
        out_shape=[jax.ShapeDtypeStruct((N, H), f32)] * 3,
    )(x, emb, pw, pb, w1a, w1b, b1)


def _gru_ln(s2, dg2, h, w2, b2, wih, whh, bih, bhh, lng, lnb):
    s = s2[0] + s2[1]
    deg = dg2[0] + dg2[1]
    agg = _matT(s, w2) + deg * b2
    gi = _matT(agg, wih) + bih
    gh = _matT(h, whh) + bhh
    r = jax.nn.sigmoid(gi[:, :H] + gh[:, :H])
    z = jax.nn.sigmoid(gi[:, H:2 * H] + gh[:, H:2 * H])
    n = jnp.tanh(gi[:, 2 * H:] + r * gh[:, 2 * H:])
    hn = h + (1.0 - z) * n + z * h
    mu = jnp.mean(hn, axis=1, keepdims=True)
    var = jnp.mean((hn - mu) ** 2, axis=1, keepdims=True)
    return (hn - mu) * lax.rsqrt(var + 1e-5) * lng + lnb


def _tc_mid_body(s2_ref, dg2_ref, h_ref, w2_ref, b2_ref, wih_ref, whh_ref,
                 bih_ref, bhh_ref, lng_ref, lnb_ref, w1a_ref, w1b_ref, b1_ref,
                 h_out, a_out, b_out):
    hl = _gru_ln(s2_ref[...], dg2_ref[...], h_ref[...], w2_ref[...],
                 b2_ref[...], wih_ref[...], whh_ref[...], bih_ref[...],
                 bhh_ref[...], lng_ref[...], lnb_ref[...])
    h_out[...] = hl
    a_out[...] = _matT(hl, w1a_ref[...]) + b1_ref[...]
    b_out[...] = _matT(hl, w1b_ref[...])


def _tc_last_body(s2_ref, dg2_ref, h_ref, w2_ref, b2_ref, wih_ref, whh_ref,
                  bih_ref, bhh_ref, lng_ref, lnb_ref, h_out):
    h_out[...] = _gru_ln(s2_ref[...], dg2_ref[...], h_ref[...], w2_ref[...],
                         b2_ref[...], wih_ref[...], whh_ref[...], bih_ref[...],
                         bhh_ref[...], lng_ref[...], lnb_ref[...])


_ROWB = 1000
_GRID = N // _ROWB


def _node_specs():
    full = lambda shape: pl.BlockSpec(shape, lambda i: tuple(0 for _ in shape))
    return [
        pl.BlockSpec((2, _ROWB, H), lambda i: (0, i, 0)),
        pl.BlockSpec((2, _ROWB, 1), lambda i: (0, i, 0)),
        pl.BlockSpec((_ROWB, H), lambda i: (i, 0)),
        full((H, H)), full((1, H)),
        full((3 * H, H)), full((3 * H, H)), full((1, 3 * H)), full((1, 3 * H)),
        full((1, H)), full((1, H)),
    ]


def _tc_mid(s2, dg2, h, w2, b2, wih, whh, bih, bhh, lng, lnb, w1a, w1b, b1):
    f32 = jnp.float32
    full = lambda shape: pl.BlockSpec(shape, lambda i: tuple(0 for _ in shape))
    row = pl.BlockSpec((_ROWB, H), lambda i: (i, 0))
    return pl.pallas_call(
        _tc_mid_body,
        grid=(_GRID,),
        in_specs=_node_specs() + [full((H, H)), full((H, H)), full((1, H))],
        out_specs=[row, row, row],
        out_shape=[jax.ShapeDtypeStruct((N, H), f32)] * 3,
    )(s2, dg2, h, w2, b2, wih, whh, bih, bhh, lng, lnb, w1a, w1b, b1)


def _tc_last(s2, dg2, h, w2, b2, wih, whh, bih, bhh, lng, lnb):
    f32 = jnp.float32
    row = pl.BlockSpec((_ROWB, H), lambda i: (i, 0))
    return pl.pallas_call(
        _tc_last_body,
        grid=(_GRID,),
        in_specs=_node_specs(),
        out_specs=row,
        out_shape=jax.ShapeDtypeStruct((N, H), f32),
    )(s2, dg2, h, w2, b2, wih, whh, bih, bhh, lng, lnb)


def _tc_fin(h, batch2, h1w, h1b, h2w, h2b):
    f32 = jnp.float32
    return pl.pallas_call(
        _fin_body2,
        out_shape=jax.ShapeDtypeStruct((NG, 1), f32),
    )(h, batch2, h1w, h1b, h2w, h2b)


def _fin_body2(h_ref, batch_ref, h1w_ref, h1b_ref, h2w_ref, h2b_ref, out_ref):
    oh = (batch_ref[...] == lax.broadcasted_iota(jnp.int32, (N, NG), 1))
    oh = oh.astype(jnp.float32)
    sums = lax.dot_general(oh, h_ref[...], (((0,), (0,)), ((), ())),
                           preferred_element_type=jnp.float32)
    ones = jnp.ones((N, 1), jnp.float32)
    cnts = lax.dot_general(oh, ones, (((0,), (0,)), ((), ())),
                           preferred_element_type=jnp.float32)
    g = sums / jnp.maximum(cnts, 1.0)
    t = _matT(g, h1w_ref[...]) + h1b_ref[...]
    g1 = t * jax.nn.sigmoid(t)
    out_ref[...] = (jnp.sum(g1 * h2w_ref[...], axis=1, keepdims=True)
                    + h2b_ref[...])


def kernel(x, edge_index, edge_attr, batch, emb, proj_W, proj_b, msg1_W,
           msg1_b, msg2_W, msg2_b, gru_Wih, gru_Whh, gru_bih, gru_bhh,
           ln_g, ln_b, head1_W, head1_b, head2_W, head2_b):
    f32 = jnp.float32
    row = edge_index[0].astype(jnp.int32)
    col = edge_index[1].astype(jnp.int32)
    pad = EPAD - E
    rg = jnp.concatenate([row, jnp.zeros((pad,), jnp.int32)]).reshape(-1, CHUNK)
    rs = jnp.concatenate([row, jnp.full((pad,), N, jnp.int32)]).reshape(-1, CHUNK)
    cg = jnp.concatenate([col, jnp.zeros((pad,), jnp.int32)]).reshape(-1, CHUNK)
    ea = jnp.concatenate([edge_attr[:, 0].astype(f32),
                          jnp.zeros((pad,), f32)]).reshape(-1, CHUNK)
    eai = lax.bitcast_convert_type(ea, jnp.int32)
    idx_pack = jnp.stack([rg, rs, cg, eai], axis=1)

    pb = proj_b.reshape(1, H)
    w1a = [msg1_W[l, :, :H] for l in range(NCONV)]
    w1b = [msg1_W[l, :, H:2 * H] for l in range(NCONV)]
    w1c = [msg1_W[l, :, 2 * H] for l in range(NCONV)]
    b1 = [msg1_b[l].reshape(1, H) for l in range(NCONV)]
    b2 = [msg2_b[l].reshape(1, H) for l in range(NCONV)]

    h, a, b = _tc0(x.astype(jnp.int32), emb, proj_W, pb, w1a[0], w1b[0], b1[0])
    dg2 = _deg_kernel(idx_pack)[:, :N, 0:1]

    for l in range(NCONV):
        s2 = _edge_kernel(a, b, idx_pack, w1c[l])[:, :N]
        args = (s2, dg2, h, msg2_W[l], b2[l], gru_Wih[l], gru_Whh[l],
                gru_bih[l].reshape(1, 3 * H), gru_bhh[l].reshape(1, 3 * H),
                ln_g[l].reshape(1, H), ln_b[l].reshape(1, H))
        if l < NCONV - 1:
            h, a, b = _tc_mid(*args, w1a[l + 1], w1b[l + 1], b1[l + 1])
        else:
            h = _tc_last(*args)

    out2 = _tc_fin(h, batch.astype(jnp.int32).reshape(N, 1), head1_W,
                   head1_b.reshape(1, H), head2_W, head2_b.reshape(1, 1))
    return out2[:, 0]


# speedup vs baseline: 4.5045x; 1.6476x over previous
import dataclasses
import functools

import jax
import jax.numpy as jnp
from jax import lax
from jax.experimental import pallas as pl
from jax.experimental.pallas import tpu as pltpu
from jax.experimental.pallas import tpu_sc as plsc

N = 10000
E = 320000
H = 128
MAXZ = 100
NG = 64
NCONV = 3

NTILES = 32
CHUNK = 80
EPT = 10080
NCHUNK = EPT // CHUNK
EPAD = EPT * NTILES
NROWPAD = 4
SROWS = 10112
NOUT = 10112

_mesh = plsc.VectorSubcoreMesh(core_axis_name="c", subcore_axis_name="s")

_sc_params = pltpu.CompilerParams()
if "needs_layout_passes" in pltpu.CompilerParams.__dataclass_fields__:
    _sc_params = dataclasses.replace(_sc_params, needs_layout_passes=False)


def _matT(a, b):
    return lax.dot_general(a, b, (((1,), (1,)), ((), ())),
                           preferred_element_type=jnp.float32)


def _edge_body(a_hbm, b_hbm, idx_hbm, w_hbm, out_hbm,
               idx0, idx1, a0, a1, b0, b1, w_v, zbuf, s_sh, g0, g1):
    ci = lax.axis_index("c")
    si = lax.axis_index("s")

    zeros16 = jnp.zeros((16,), jnp.float32)

    @pl.loop(0, 8)
    def _(r):
        for j in range(H // 16):
            zbuf[r, pl.ds(j * 16, 16)] = zeros16

    @pl.loop(0, SROWS // 16 // 8)
    def _(k):
        pltpu.sync_copy(zbuf, s_sh.at[pl.ds(si * (SROWS // 16) + k * 8, 8), :])

    pltpu.sync_copy(w_hbm, w_v)
    plsc.subcore_barrier()

    wblocks = [w_v[pl.ds(j * 16, 16)] for j in range(8)]
    wid = ci * 16 + si

    three16 = jnp.full((16,), 3, jnp.int32)

    def compute(idx_v, a_buf, b_buf):
        @plsc.parallel_loop(0, CHUNK, unroll=4)
        def _(e):
            eidx = jnp.full((16,), e, jnp.int32)
            ea16 = plsc.bitcast(plsc.load_gather(idx_v, [three16, eidx]),
                                jnp.float32)
            for jb in range(8):
                a = a_buf[e, pl.ds(jb * 16, 16)]
                b = b_buf[e, pl.ds(jb * 16, 16)]
                t = a + b + ea16 * wblocks[jb]
                a_buf[e, pl.ds(jb * 16, 16)] = t / (1.0 + jnp.exp(-t))

    def prefetch(idx_v, a_buf, b_buf, sem, j):
        pltpu.sync_copy(idx_hbm.at[j], idx_v)
        pltpu.async_copy(a_hbm.at[idx_v.at[0]], a_buf, sem)
        pltpu.async_copy(b_hbm.at[idx_v.at[2]], b_buf, sem)

    def drain(idx_v, a_buf, b_buf, sem):
        pltpu.make_async_copy(a_hbm.at[idx_v.at[0]], a_buf, sem).wait()
        pltpu.make_async_copy(b_hbm.at[idx_v.at[2]], b_buf, sem).wait()

    def half(idx_v, a_buf, b_buf, sem, jnext):
        drain(idx_v, a_buf, b_buf, sem)
        compute(idx_v, a_buf, b_buf)
        pltpu.sync_copy(a_buf, s_sh.at[idx_v.at[1]], add=True)
        prefetch(idx_v, a_buf, b_buf, sem, jnext)

    base = wid * NCHUNK
    prefetch(idx0, a0, b0, g0, base)
    prefetch(idx1, a1, b1, g1, base + 1)

    @pl.loop(0, NCHUNK // 2)
    def _(p):
        j = base + 2 * p
        half(idx0, a0, b0, g0, j + 2)
        half(idx1, a1, b1, g1, j + 3)

    drain(idx0, a0, b0, g0)
    drain(idx1, a1, b1, g1)
    plsc.subcore_barrier()
    rows = NOUT // 16
    pltpu.sync_copy(s_sh.at[pl.ds(si * rows, rows), :],
                    out_hbm.at[ci, pl.ds(si * rows, rows), :])


_edge_kernel = functools.partial(
    pl.kernel,
    out_type=jax.ShapeDtypeStruct((2, NOUT, H), jnp.float32),
    mesh=_mesh,
    compiler_params=_sc_params,
    scratch_types=[
        pltpu.VMEM((4, CHUNK), jnp.int32),
        pltpu.VMEM((4, CHUNK), jnp.int32),
        pltpu.VMEM((CHUNK, H), jnp.float32),
        pltpu.VMEM((CHUNK, H), jnp.float32),
        pltpu.VMEM((CHUNK, H), jnp.float32),
        pltpu.VMEM((CHUNK, H), jnp.float32),
        pltpu.VMEM((H,), jnp.float32),
        pltpu.VMEM((8, H), jnp.float32),
        pltpu.VMEM_SHARED((SROWS, H), jnp.float32),
        pltpu.SemaphoreType.DMA,
        pltpu.SemaphoreType.DMA,
    ],
)(_edge_body)


def _deg_body(idx_hbm, out_hbm, idx_v, ones_buf, zbuf, s_sh):
    ci = lax.axis_index("c")
    si = lax.axis_index("s")
    zeros16 = jnp.zeros((16,), jnp.float32)
    ones16 = jnp.ones((16,), jnp.float32)

    @pl.loop(0, CHUNK)
    def _(r):
        for j in range(H // 16):
            ones_buf[r, pl.ds(j * 16, 16)] = ones16

    @pl.loop(0, 8)
    def _(r):
        for j in range(H // 16):
            zbuf[r, pl.ds(j * 16, 16)] = zeros16

    @pl.loop(0, SROWS // 16 // 8)
    def _(k):
        pltpu.sync_copy(zbuf, s_sh.at[pl.ds(si * (SROWS // 16) + k * 8, 8), :])

    plsc.subcore_barrier()
    wid = ci * 16 + si

    @pl.loop(0, NCHUNK)
    def _(k):
        j = wid * NCHUNK + k
        pltpu.sync_copy(idx_hbm.at[j], idx_v)
        pltpu.sync_copy(ones_buf, s_sh.at[idx_v.at[1]], add=True)

    plsc.subcore_barrier()
    rows = NOUT // 16
    pltpu.sync_copy(s_sh.at[pl.ds(si * rows, rows), :],
                    out_hbm.at[ci, pl.ds(si * rows, rows), :])


_deg_kernel = functools.partial(
    pl.kernel,
    out_type=jax.ShapeDtypeStruct((2, NOUT, H), jnp.float32),
    mesh=_mesh,
    compiler_params=_sc_params,
    scratch_types=[
        pltpu.VMEM((4, CHUNK), jnp.int32),
        pltpu.VMEM((CHUNK, H), jnp.float32),
        pltpu.VMEM((8, H), jnp.float32),
        pltpu.VMEM_SHARED((SROWS, H), jnp.float32),
    ],
)(_deg_body)


def _tc0_body(x_ref, emb_ref, pw_ref, pb_ref, w1a_ref, w1b_ref, b1_ref,
              h_ref, a_ref, b_ref):
    oh = (x_ref[...] == lax.broadcasted_iota(jnp.int32, (N, MAXZ + 1), 1))
    oh = oh.astype(jnp.float32)
    he = jnp.dot(oh, emb_ref[...], preferred_element_type=jnp.float32)
    h = _matT(he, pw_ref[...]) + pb_ref[...]
    h_ref[...] = h
    a_ref[...] = _matT(h, w1a_ref[...]) + b1_ref[...]
    b_ref[...] = _matT(h, w1b_ref[...])


def _tc0(x, emb, pw, pb, w1a, w1b, b1):
    f32 = jnp.float32
    return pl.pallas_call(
        _tc0_body,
        out_shape=[jax.ShapeDtypeStruct((N, H), f32)] * 3,
    )(x, emb, pw, pb, w1a, w1b, b1)


def _gru_ln(s2, dg2, h, w2, b2, wih, whh, bih, bhh, lng, lnb):
    s = s2[0] + s2[1]
    deg = dg2[0] + dg2[1]
    agg = _matT(s, w2) + deg * b2
    gi = _matT(agg, wih) + bih
    gh = _matT(h, whh) + bhh
    r = jax.nn.sigmoid(gi[:, :H] + gh[:, :H])
    z = jax.nn.sigmoid(gi[:, H:2 * H] + gh[:, H:2 * H])
    n = jnp.tanh(gi[:, 2 * H:] + r * gh[:, 2 * H:])
    hn = h + (1.0 - z) * n + z * h
    mu = jnp.mean(hn, axis=1, keepdims=True)
    var = jnp.mean((hn - mu) ** 2, axis=1, keepdims=True)
    return (hn - mu) * lax.rsqrt(var + 1e-5) * lng + lnb


def _tc_mid_body(s2_ref, dg2_ref, h_ref, w2_ref, b2_ref, wih_ref, whh_ref,
                 bih_ref, bhh_ref, lng_ref, lnb_ref, w1a_ref, w1b_ref, b1_ref,
                 h_out, a_out, b_out):
    hl = _gru_ln(s2_ref[...], dg2_ref[...], h_ref[...], w2_ref[...],
                 b2_ref[...], wih_ref[...], whh_ref[...], bih_ref[...],
                 bhh_ref[...], lng_ref[...], lnb_ref[...])
    h_out[...] = hl
    a_out[...] = _matT(hl, w1a_ref[...]) + b1_ref[...]
    b_out[...] = _matT(hl, w1b_ref[...])


def _tc_last_body(s2_ref, dg2_ref, h_ref, w2_ref, b2_ref, wih_ref, whh_ref,
                  bih_ref, bhh_ref, lng_ref, lnb_ref, h_out):
    h_out[...] = _gru_ln(s2_ref[...], dg2_ref[...], h_ref[...], w2_ref[...],
                         b2_ref[...], wih_ref[...], whh_ref[...], bih_ref[...],
                         bhh_ref[...], lng_ref[...], lnb_ref[...])


_ROWB = 1000
_GRID = N // _ROWB


def _node_specs():
    full = lambda shape: pl.BlockSpec(shape, lambda i: tuple(0 for _ in shape))
    return [
        pl.BlockSpec((2, _ROWB, H), lambda i: (0, i, 0)),
        pl.BlockSpec((2, _ROWB, 1), lambda i: (0, i, 0)),
        pl.BlockSpec((_ROWB, H), lambda i: (i, 0)),
        full((H, H)), full((1, H)),
        full((3 * H, H)), full((3 * H, H)), full((1, 3 * H)), full((1, 3 * H)),
        full((1, H)), full((1, H)),
    ]


def _tc_mid(s2, dg2, h, w2, b2, wih, whh, bih, bhh, lng, lnb, w1a, w1b, b1):
    f32 = jnp.float32
    full = lambda shape: pl.BlockSpec(shape, lambda i: tuple(0 for _ in shape))
    row = pl.BlockSpec((_ROWB, H), lambda i: (i, 0))
    return pl.pallas_call(
        _tc_mid_body,
        grid=(_GRID,),
        in_specs=_node_specs() + [full((H, H)), full((H, H)), full((1, H))],
        out_specs=[row, row, row],
        out_shape=[jax.ShapeDtypeStruct((N, H), f32)] * 3,
    )(s2, dg2, h, w2, b2, wih, whh, bih, bhh, lng, lnb, w1a, w1b, b1)


def _tc_last(s2, dg2, h, w2, b2, wih, whh, bih, bhh, lng, lnb):
    f32 = jnp.float32
    row = pl.BlockSpec((_ROWB, H), lambda i: (i, 0))
    return pl.pallas_call(
        _tc_last_body,
        grid=(_GRID,),
        in_specs=_node_specs(),
        out_specs=row,
        out_shape=jax.ShapeDtypeStruct((N, H), f32),
    )(s2, dg2, h, w2, b2, wih, whh, bih, bhh, lng, lnb)


def _tc_fin(h, batch2, h1w, h1b, h2w, h2b):
    f32 = jnp.float32
    return pl.pallas_call(
        _fin_body2,
        out_shape=jax.ShapeDtypeStruct((NG, 1), f32),
    )(h, batch2, h1w, h1b, h2w, h2b)


def _fin_body2(h_ref, batch_ref, h1w_ref, h1b_ref, h2w_ref, h2b_ref, out_ref):
    oh = (batch_ref[...] == lax.broadcasted_iota(jnp.int32, (N, NG), 1))
    oh = oh.astype(jnp.float32)
    sums = lax.dot_general(oh, h_ref[...], (((0,), (0,)), ((), ())),
                           preferred_element_type=jnp.float32)
    ones = jnp.ones((N, 1), jnp.float32)
    cnts = lax.dot_general(oh, ones, (((0,), (0,)), ((), ())),
                           preferred_element_type=jnp.float32)
    g = sums / jnp.maximum(cnts, 1.0)
    t = _matT(g, h1w_ref[...]) + h1b_ref[...]
    g1 = t * jax.nn.sigmoid(t)
    out_ref[...] = (jnp.sum(g1 * h2w_ref[...], axis=1, keepdims=True)
                    + h2b_ref[...])


def kernel(x, edge_index, edge_attr, batch, emb, proj_W, proj_b, msg1_W,
           msg1_b, msg2_W, msg2_b, gru_Wih, gru_Whh, gru_bih, gru_bhh,
           ln_g, ln_b, head1_W, head1_b, head2_W, head2_b):
    f32 = jnp.float32
    row = edge_index[0].astype(jnp.int32)
    col = edge_index[1].astype(jnp.int32)
    pad = EPAD - E
    rg = jnp.concatenate([row, jnp.zeros((pad,), jnp.int32)]).reshape(-1, CHUNK)
    rs = jnp.concatenate([row, jnp.full((pad,), N, jnp.int32)]).reshape(-1, CHUNK)
    cg = jnp.concatenate([col, jnp.zeros((pad,), jnp.int32)]).reshape(-1, CHUNK)
    ea = jnp.concatenate([edge_attr[:, 0].astype(f32),
                          jnp.zeros((pad,), f32)]).reshape(-1, CHUNK)
    eai = lax.bitcast_convert_type(ea, jnp.int32)
    idx_pack = jnp.stack([rg, rs, cg, eai], axis=1)
    idx_pack = jnp.pad(idx_pack, ((0, NROWPAD), (0, 0), (0, 0)))

    pb = proj_b.reshape(1, H)
    w1a = [msg1_W[l, :, :H] for l in range(NCONV)]
    w1b = [msg1_W[l, :, H:2 * H] for l in range(NCONV)]
    w1c = [msg1_W[l, :, 2 * H] for l in range(NCONV)]
    b1 = [msg1_b[l].reshape(1, H) for l in range(NCONV)]
    b2 = [msg2_b[l].reshape(1, H) for l in range(NCONV)]

    h, a, b = _tc0(x.astype(jnp.int32), emb, proj_W, pb, w1a[0], w1b[0], b1[0])
    dg2 = _deg_kernel(idx_pack)[:, :N, 0:1]

    for l in range(NCONV):
        s2 = _edge_kernel(a, b, idx_pack, w1c[l])[:, :N]
        args = (s2, dg2, h, msg2_W[l], b2[l], gru_Wih[l], gru_Whh[l],
                gru_bih[l].reshape(1, 3 * H), gru_bhh[l].reshape(1, 3 * H),
                ln_g[l].reshape(1, H), ln_b[l].reshape(1, H))
        if l < NCONV - 1:
            h, a, b = _tc_mid(*args, w1a[l + 1], w1b[l + 1], b1[l + 1])
        else:
            h = _tc_last(*args)

    out2 = _tc_fin(h, batch.astype(jnp.int32).reshape(N, 1), head1_W,
                   head1_b.reshape(1, H), head2_W, head2_b.reshape(1, 1))
    return out2[:, 0]
